# baseline (speedup 1.0000x reference)
import jax
import jax.numpy as jnp
from jax import lax
from jax.experimental import pallas as pl
from jax.experimental.pallas import tpu as pltpu
from jax.experimental.pallas import tpu_sc as plsc

B, N, K, D = 4, 2048, 8192, 3
BN = B * N
NC, NS, L = 2, 16, 16
NW = NC * NS
G = 4
KU = 4
CHUNKS = K // L
SP = 2560
PWC = SP // NW
TCP = BN - SP
PN = 256


def _sc_body(pts_hbm, keys_hbm, kid_hbm, out_hbm, pts_v, keys_v, kid_v,
             ids_v, rot_v, rot_i):
    wid = lax.axis_index("s") * NC + lax.axis_index("c")
    pltpu.sync_copy(keys_hbm, keys_v)
    pltpu.sync_copy(kid_hbm, kid_v)
    pltpu.sync_copy(pts_hbm.at[pl.ds((TCP + wid * PWC) * D, PWC * D)], pts_v)
    iota = lax.iota(jnp.int32, L)
    inf = jnp.full((L,), jnp.inf, jnp.float32)
    zeros = jnp.zeros((L,), jnp.int32)

    def group_fn(g, _):
        res = zeros
        vs = [pts_v[pl.ds(g * (D * L) + v * L, L)] for v in range(D)]
        for sb in range(L // G):
            px, py, pz = [], [], []
            for q in range(G):
                j = D * (sb * G + q)
                px.append(jnp.full((L,), vs[j // L][j % L], jnp.float32))
                py.append(jnp.full((L,), vs[(j + 1) // L][(j + 1) % L],
                                   jnp.float32))
                pz.append(jnp.full((L,), vs[(j + 2) // L][(j + 2) % L],
                                   jnp.float32))

            def chunk_fn(c, carry):
                best = list(carry[0])
                bidx = list(carry[1])
                for u in range(KU):
                    off = (c * KU + u) * L
                    kx = keys_v[pl.ds(off, L)]
                    ky = keys_v[pl.ds(K + off, L)]
                    kz = keys_v[pl.ds(2 * K + off, L)]
                    kidx = kid_v[pl.ds(off, L)]
                    for q in range(G):
                        dx = px[q] - kx
                        dy = py[q] - ky
                        dz = pz[q] - kz
                        dist = dx * dx + dy * dy + dz * dz
                        lt = dist < best[q]
                        best[q] = jnp.where(lt, dist, best[q])
                        bidx[q] = jnp.where(lt, kidx, bidx[q])
                return tuple(best), tuple(bidx)

            best, bidx = lax.fori_loop(
                0, CHUNKS // KU, chunk_fn,
                (tuple(inf for _ in range(G)), tuple(zeros for _ in range(G))))

            for q in range(G):
                b, bi = best[q], bidx[q]
                for r in (8, 4, 2, 1):
                    rot_v[pl.ds(0, L)] = b
                    rot_v[pl.ds(L, L)] = b
                    rot_i[pl.ds(0, L)] = bi
                    rot_i[pl.ds(L, L)] = bi
                    rb = rot_v[pl.ds(r, L)]
                    ri = rot_i[pl.ds(r, L)]
                    take = (rb < b) | ((rb == b) & (ri < bi))
                    b = jnp.where(take, rb, b)
                    bi = jnp.where(take, ri, bi)
                res = jnp.where(iota == sb * G + q, bi, res)
        ids_v[pl.ds(g * L, L)] = res
        return 0

    lax.fori_loop(0, PWC // L, group_fn, 0)
    pltpu.sync_copy(ids_v, out_hbm.at[pl.ds(wid * PWC, PWC)])


def _tc_body(pts_ref, keys_t_ref, out_ref):
    px = pts_ref[:, 0:1]
    py = pts_ref[:, 1:2]
    pz = pts_ref[:, 2:3]
    kx = keys_t_ref[0:1, :]
    ky = keys_t_ref[1:2, :]
    kz = keys_t_ref[2:3, :]
    dx = px - kx
    dy = py - ky
    dz = pz - kz
    dist = dx * dx + dy * dy + dz * dz
    out_ref[:, :] = jnp.argmin(dist, axis=1, keepdims=True).astype(jnp.int32)


def kernel(pointcloud, keys, table):
    del table
    pts_flat = pointcloud.reshape(-1)
    keys_flat = jnp.concatenate([keys[:, 0], keys[:, 1], keys[:, 2]])
    keys_t = keys_flat.reshape(D, K)

    ids_sc = pl.kernel(
        _sc_body,
        out_type=jax.ShapeDtypeStruct((SP,), jnp.int32),
        mesh=plsc.VectorSubcoreMesh(core_axis_name="c", subcore_axis_name="s"),
        scratch_types=[
            pltpu.VMEM((PWC * D,), jnp.float32),
            pltpu.VMEM((D * K,), jnp.float32),
            pltpu.VMEM((K,), jnp.int32),
            pltpu.VMEM((PWC,), jnp.int32),
            pltpu.VMEM((2 * L,), jnp.float32),
            pltpu.VMEM((2 * L,), jnp.int32),
        ],
    )(pts_flat, keys_flat, jnp.arange(K, dtype=jnp.int32))

    ids_tc = pl.pallas_call(
        _tc_body,
        grid=(TCP // PN,),
        in_specs=[
            pl.BlockSpec((PN, D), lambda i: (i, 0)),
            pl.BlockSpec((D, K), lambda i: (0, 0)),
        ],
        out_specs=pl.BlockSpec((PN, 1), lambda i: (i, 0)),
        out_shape=jax.ShapeDtypeStruct((TCP, 1), jnp.int32),
    )(pointcloud.reshape(BN, D)[:TCP], keys_t)

    ids = jnp.concatenate([ids_tc.reshape(-1), ids_sc])
    return (ids.reshape(B, N, 1), pointcloud)


# speedup vs baseline: 1.0093x; 1.0093x over previous
import jax
import jax.numpy as jnp
from jax import lax
from jax.experimental import pallas as pl
from jax.experimental.pallas import tpu as pltpu
from jax.experimental.pallas import tpu_sc as plsc

B, N, K, D = 4, 2048, 8192, 3
BN = B * N
NC, NS, L = 2, 16, 16
NW = NC * NS
G = 4
KU = 4
CHUNKS = K // L
SP = 2560
PWC = SP // NW
TCP = BN - SP
PN = 256


def _sc_body(pts_hbm, keys_hbm, out_hbm, pts_v, keys_v, ids_v, rot_v, rot_i):
    wid = lax.axis_index("s") * NC + lax.axis_index("c")
    pltpu.sync_copy(keys_hbm, keys_v)
    pltpu.sync_copy(pts_hbm.at[pl.ds((TCP + wid * PWC) * D, PWC * D)], pts_v)
    iota = lax.iota(jnp.int32, L)
    inf = jnp.full((L,), jnp.inf, jnp.float32)
    zeros = jnp.zeros((L,), jnp.int32)

    def group_fn(g, _):
        res = zeros
        vs = [pts_v[pl.ds(g * (D * L) + v * L, L)] for v in range(D)]
        for sb in range(L // G):
            px, py, pz = [], [], []
            for q in range(G):
                j = D * (sb * G + q)
                px.append(jnp.full((L,), vs[j // L][j % L], jnp.float32))
                py.append(jnp.full((L,), vs[(j + 1) // L][(j + 1) % L],
                                   jnp.float32))
                pz.append(jnp.full((L,), vs[(j + 2) // L][(j + 2) % L],
                                   jnp.float32))

            def chunk_fn(c, carry):
                best = list(carry[0])
                bidx = list(carry[1])
                for u in range(KU):
                    off = (c * KU + u) * L
                    kx = keys_v[pl.ds(off, L)]
                    ky = keys_v[pl.ds(K + off, L)]
                    kz = keys_v[pl.ds(2 * K + off, L)]
                    kidx = iota + off
                    for q in range(G):
                        dx = px[q] - kx
                        dy = py[q] - ky
                        dz = pz[q] - kz
                        dist = dx * dx + dy * dy + dz * dz
                        lt = dist < best[q]
                        best[q] = jnp.where(lt, dist, best[q])
                        bidx[q] = jnp.where(lt, kidx, bidx[q])
                return tuple(best), tuple(bidx)

            best, bidx = lax.fori_loop(
                0, CHUNKS // KU, chunk_fn,
                (tuple(inf for _ in range(G)), tuple(zeros for _ in range(G))))

            for q in range(G):
                b, bi = best[q], bidx[q]
                for r in (8, 4, 2, 1):
                    rot_v[pl.ds(0, L)] = b
                    rot_v[pl.ds(L, L)] = b
                    rot_i[pl.ds(0, L)] = bi
                    rot_i[pl.ds(L, L)] = bi
                    rb = rot_v[pl.ds(r, L)]
                    ri = rot_i[pl.ds(r, L)]
                    take = (rb < b) | ((rb == b) & (ri < bi))
                    b = jnp.where(take, rb, b)
                    bi = jnp.where(take, ri, bi)
                res = jnp.where(iota == sb * G + q, bi, res)
        ids_v[pl.ds(g * L, L)] = res
        return 0

    lax.fori_loop(0, PWC // L, group_fn, 0)
    pltpu.sync_copy(ids_v, out_hbm.at[pl.ds(wid * PWC, PWC)])


def _tc_body(pts_ref, keys_t_ref, out_ref):
    px = pts_ref[:, 0:1]
    py = pts_ref[:, 1:2]
    pz = pts_ref[:, 2:3]
    kx = keys_t_ref[0:1, :]
    ky = keys_t_ref[1:2, :]
    kz = keys_t_ref[2:3, :]
    dx = px - kx
    dy = py - ky
    dz = pz - kz
    dist = dx * dx + dy * dy + dz * dz
    out_ref[:, :] = jnp.argmin(dist, axis=1, keepdims=True).astype(jnp.int32)


def kernel(pointcloud, keys, table):
    del table
    pts_flat = pointcloud.reshape(-1)
    keys_flat = jnp.concatenate([keys[:, 0], keys[:, 1], keys[:, 2]])
    keys_t = keys_flat.reshape(D, K)

    ids_sc = pl.kernel(
        _sc_body,
        out_type=jax.ShapeDtypeStruct((SP,), jnp.int32),
        mesh=plsc.VectorSubcoreMesh(core_axis_name="c", subcore_axis_name="s"),
        scratch_types=[
            pltpu.VMEM((PWC * D,), jnp.float32),
            pltpu.VMEM((D * K,), jnp.float32),
            pltpu.VMEM((PWC,), jnp.int32),
            pltpu.VMEM((2 * L,), jnp.float32),
            pltpu.VMEM((2 * L,), jnp.int32),
        ],
    )(pts_flat, keys_flat)

    ids_tc = pl.pallas_call(
        _tc_body,
        grid=(TCP // PN,),
        in_specs=[
            pl.BlockSpec((PN, D), lambda i: (i, 0)),
            pl.BlockSpec((D, K), lambda i: (0, 0)),
        ],
        out_specs=pl.BlockSpec((PN, 1), lambda i: (i, 0)),
        out_shape=jax.ShapeDtypeStruct((TCP, 1), jnp.int32),
    )(pointcloud.reshape(BN, D)[:TCP], keys_t)

    ids = jnp.concatenate([ids_tc.reshape(-1), ids_sc])
    return (ids.reshape(B, N, 1), pointcloud)
